# Initial kernel scaffold; baseline (speedup 1.0000x reference)
import jax
import jax.numpy as jnp
from jax.experimental import pallas as pl


def kernel(pos0, pos1, pos2, x0, x1, x2, batch0, batch1, batch2, W_up1, b_up1, W_res1a, b_res1a, W_res1b, b_res1b, W_up2, b_up2, W_res2a, b_res2a, W_res2b, b_res2b):
    raise NotImplementedError("write your pallas kernel here")


# speedup vs baseline: 12.5527x; 12.5527x over previous
import functools

import jax
import jax.numpy as jnp
from jax import lax
from jax.experimental import pallas as pl

_BIG = 3.4e38
_PEN = 1e9


def _level_body(pu_ref, bu_ref, posT_ref, bc_ref, x_ref, xs_ref,
                w_top_ref, w_bot_ref, b_up_ref, wa_ref, ba_ref,
                wb_ref, bb_ref, out_ref, *, n_coarse):
    R = pu_ref.shape[0]
    db = bu_ref[...] - bc_ref[...]
    d2 = db * db * _PEN
    for c in range(3):
        diff = pu_ref[:, c:c + 1] - posT_ref[c:c + 1, :]
        d2 = d2 + diff * diff

    cols = lax.broadcasted_iota(jnp.int32, (R, n_coarse), 1)
    wmat = jnp.zeros((R, n_coarse), jnp.float32)
    den = jnp.zeros((R, 1), jnp.float32)
    for _ in range(3):
        m = jnp.min(d2, axis=1, keepdims=True)
        idx = jnp.min(jnp.where(d2 == m, cols, n_coarse), axis=1,
                      keepdims=True)
        sel = cols == idx
        w = 1.0 / jnp.maximum(m, 1e-16)
        wmat = jnp.where(sel, w, wmat)
        den = den + w
        d2 = jnp.where(sel, _BIG, d2)

    up = jnp.dot(wmat, x_ref[...],
                 preferred_element_type=jnp.float32) / den
    xi = jnp.dot(xs_ref[...], w_top_ref[...], preferred_element_type=jnp.float32)
    xi = xi + jnp.dot(up, w_bot_ref[...], preferred_element_type=jnp.float32)
    xi = jax.nn.relu(xi + b_up_ref[...])
    h = jax.nn.relu(jnp.dot(xi, wa_ref[...], preferred_element_type=jnp.float32)
                    + ba_ref[...])
    out_ref[...] = xi + jnp.dot(h, wb_ref[...],
                                preferred_element_type=jnp.float32) + bb_ref[...]


def _level(pos_up, batch_up, pos, batch, x, x_skip,
           W_up, b_up, Wa, ba, Wb, bb, row_tile):
    n_up = pos_up.shape[0]
    n_coarse = pos.shape[0]
    c_in = x.shape[1]
    c_skip = x_skip.shape[1]
    c_out = Wa.shape[0]
    grid = (n_up // row_tile,)

    posT = pos.T
    bu = batch_up.astype(jnp.float32)[:, None]
    bc = batch.astype(jnp.float32)[None, :]
    w_top = W_up[:c_skip]
    w_bot = W_up[c_skip:]

    row_spec = lambda cols_: pl.BlockSpec((row_tile, cols_), lambda i: (i, 0))
    full = lambda a: pl.BlockSpec(a.shape, lambda i: (0,) * a.ndim)

    fn = pl.pallas_call(
        functools.partial(_level_body, n_coarse=n_coarse),
        grid=grid,
        in_specs=[
            row_spec(3),
            row_spec(1),
            full(posT),
            full(bc),
            full(x),
            row_spec(c_skip),
            full(w_top), full(w_bot), full(b_up[None, :]),
            full(Wa), full(ba[None, :]),
            full(Wb), full(bb[None, :]),
        ],
        out_specs=row_spec(c_out),
        out_shape=jax.ShapeDtypeStruct((n_up, c_out), jnp.float32),
    )
    return fn(pos_up, bu, posT, bc, x, x_skip,
              w_top, w_bot, b_up[None, :], Wa, ba[None, :], Wb, bb[None, :])


def kernel(pos0, pos1, pos2, x0, x1, x2, batch0, batch1, batch2,
           W_up1, b_up1, W_res1a, b_res1a, W_res1b, b_res1b,
           W_up2, b_up2, W_res2a, b_res2a, W_res2b, b_res2b):
    xi1 = _level(pos1, batch1, pos0, batch0, x0, x1,
                 W_up1, b_up1, W_res1a, b_res1a, W_res1b, b_res1b,
                 row_tile=256)
    xi2 = _level(pos2, batch2, pos1, batch1, xi1, x2,
                 W_up2, b_up2, W_res2a, b_res2a, W_res2b, b_res2b,
                 row_tile=256)
    return xi2


# speedup vs baseline: 15.3502x; 1.2229x over previous
import functools

import jax
import jax.numpy as jnp
from jax import lax
from jax.experimental import pallas as pl

_BIG = 3.4e38
_PEN = 1e9


def _level_body(pu_ref, bu_ref, posT_ref, bc_ref, x_ref, xs_ref,
                w_top_ref, w_bot_ref, b_up_ref, wa_ref, ba_ref,
                wb_ref, bb_ref, out_ref, *, n_coarse):
    R = pu_ref.shape[0]
    pu = pu_ref[...]
    pu2 = jnp.sum(pu * pu, axis=1, keepdims=True)
    posT = posT_ref[...]
    p2 = jnp.sum(posT * posT, axis=0, keepdims=True)
    dot = jnp.dot(pu, posT, preferred_element_type=jnp.float32)
    db = bu_ref[...] - bc_ref[...]
    d2 = jnp.maximum(pu2 - 2.0 * dot + p2, 0.0) + db * db * _PEN

    cols = lax.broadcasted_iota(jnp.int32, (R, n_coarse), 1)
    key = (lax.bitcast_convert_type(d2, jnp.int32) & ~4095) | cols
    big_key = jnp.int32(0x7F7FFFFF)

    wmat = jnp.zeros((R, n_coarse), jnp.float32)
    den = jnp.zeros((R, 1), jnp.float32)
    for _ in range(3):
        m = jnp.min(key, axis=1, keepdims=True)
        sel = key == m
        d2_k = lax.bitcast_convert_type(m & ~4095, jnp.float32)
        w = 1.0 / jnp.maximum(d2_k, 1e-16)
        wmat = jnp.where(sel, w, wmat)
        den = den + w
        key = jnp.where(sel, big_key, key)

    up = jnp.dot(wmat, x_ref[...],
                 preferred_element_type=jnp.float32) / den
    xi = jnp.dot(xs_ref[...], w_top_ref[...], preferred_element_type=jnp.float32)
    xi = xi + jnp.dot(up, w_bot_ref[...], preferred_element_type=jnp.float32)
    xi = jax.nn.relu(xi + b_up_ref[...])
    h = jax.nn.relu(jnp.dot(xi, wa_ref[...], preferred_element_type=jnp.float32)
                    + ba_ref[...])
    out_ref[...] = xi + jnp.dot(h, wb_ref[...],
                                preferred_element_type=jnp.float32) + bb_ref[...]


def _level(pos_up, batch_up, pos, batch, x, x_skip,
           W_up, b_up, Wa, ba, Wb, bb, row_tile):
    n_up = pos_up.shape[0]
    n_coarse = pos.shape[0]
    c_in = x.shape[1]
    c_skip = x_skip.shape[1]
    c_out = Wa.shape[0]
    grid = (n_up // row_tile,)

    posT = pos.T
    bu = batch_up.astype(jnp.float32)[:, None]
    bc = batch.astype(jnp.float32)[None, :]
    w_top = W_up[:c_skip]
    w_bot = W_up[c_skip:]

    row_spec = lambda cols_: pl.BlockSpec((row_tile, cols_), lambda i: (i, 0))
    full = lambda a: pl.BlockSpec(a.shape, lambda i: (0,) * a.ndim)

    fn = pl.pallas_call(
        functools.partial(_level_body, n_coarse=n_coarse),
        grid=grid,
        in_specs=[
            row_spec(3),
            row_spec(1),
            full(posT),
            full(bc),
            full(x),
            row_spec(c_skip),
            full(w_top), full(w_bot), full(b_up[None, :]),
            full(Wa), full(ba[None, :]),
            full(Wb), full(bb[None, :]),
        ],
        out_specs=row_spec(c_out),
        out_shape=jax.ShapeDtypeStruct((n_up, c_out), jnp.float32),
    )
    return fn(pos_up, bu, posT, bc, x, x_skip,
              w_top, w_bot, b_up[None, :], Wa, ba[None, :], Wb, bb[None, :])


def kernel(pos0, pos1, pos2, x0, x1, x2, batch0, batch1, batch2,
           W_up1, b_up1, W_res1a, b_res1a, W_res1b, b_res1b,
           W_up2, b_up2, W_res2a, b_res2a, W_res2b, b_res2b):
    xi1 = _level(pos1, batch1, pos0, batch0, x0, x1,
                 W_up1, b_up1, W_res1a, b_res1a, W_res1b, b_res1b,
                 row_tile=256)
    xi2 = _level(pos2, batch2, pos1, batch1, xi1, x2,
                 W_up2, b_up2, W_res2a, b_res2a, W_res2b, b_res2b,
                 row_tile=256)
    return xi2
